# Initial kernel scaffold; baseline (speedup 1.0000x reference)
import jax
import jax.numpy as jnp
from jax.experimental import pallas as pl


def kernel(feat0, feat1, feat2, feat3, points):
    raise NotImplementedError("write your pallas kernel here")


# speedup vs baseline: 3.4091x; 3.4091x over previous
import functools

import jax
import jax.numpy as jnp
from jax import lax
from jax.experimental import pallas as pl
from jax.experimental.pallas import tpu as pltpu
from jax.experimental.pallas import tpu_sc as plsc

_NC = 2
_NS = 16
_NW = _NC * _NS
_L = 16

_SIZES = (56, 28, 14, 7)
_CHANS = (64, 128, 256, 512)
_FEAT_D = 960
_OFFS = (0, 64, 192, 448)


def _sc_body(nblk_total, tq0, tq1, tq2, tq3, xs_hbm, ys_hbm, zs_hbm, out_hbm,
             xs_v, ys_v, zs_v, idx0, idx1, idx2, idx3, r0, r1, r2, r3,
             out_buf, sem):
  tqs = (tq0, tq1, tq2, tq3)
  idxs = (idx0, idx1, idx2, idx3)
  rows = (r0, r1, r2, r3)

  wid = lax.axis_index("s") * _NC + lax.axis_index("c")
  q, r = nblk_total // _NW, nblk_total % _NW
  base_blk = wid * q + jnp.minimum(wid, r)
  n_blk = q + jnp.where(wid < r, 1, 0)

  lanes = lax.iota(jnp.int32, _L)
  zero = jnp.zeros((_L,), jnp.float32)

  def block_body(i, carry):
    gb = (base_blk + i) * _L
    pltpu.sync_copy(xs_hbm.at[pl.ds(gb, _L)], xs_v)
    pltpu.sync_copy(ys_hbm.at[pl.ds(gb, _L)], ys_v)
    pltpu.sync_copy(zs_hbm.at[pl.ds(gb, _L)], zs_v)

    xv = xs_v[...]
    yv = ys_v[...]
    zv = zs_v[...]

    h = 248.0 * (yv / zv) + 111.5
    w = 248.0 * (xv / (-zv)) + 111.5
    h = jnp.clip(h, 0.0, 223.0)
    w = jnp.clip(w, 0.0, 223.0)

    wvecs = []
    for m in range(4):
      s = _SIZES[m]
      x = h * (s / 224.0)
      y = w * (s / 224.0)
      x1i = x.astype(jnp.int32)
      x1f = x1i.astype(jnp.float32)
      x2f = jnp.minimum(jnp.where(x > x1f, x1f + 1.0, x1f), float(s - 1))
      y1i = y.astype(jnp.int32)
      y1f = y1i.astype(jnp.float32)
      y2f = jnp.minimum(jnp.where(y > y1f, y1f + 1.0, y1f), float(s - 1))

      dx0 = x2f - x
      dx1 = x - x1f
      dy0 = y2f - y
      dy1 = y - y1f
      w11 = dx0 * dy0
      w12 = dx0 * dy1
      w21 = dx1 * dy0
      w22 = dx1 * dy1
      ysame = y2f == y1f
      xsame = x2f == x1f
      a0 = w11 + jnp.where(ysame, w12, zero)
      a1 = jnp.where(ysame, zero, w12)
      a2 = w21 + jnp.where(ysame, w22, zero)
      a3 = jnp.where(ysame, zero, w22)
      wvecs.append((
          a0 + jnp.where(xsame, a2, zero),
          a1 + jnp.where(xsame, a3, zero),
          jnp.where(xsame, zero, a2),
          jnp.where(xsame, zero, a3),
      ))
      idxs[m][...] = x1i * s + y1i

    copies = [
        pltpu.async_copy(tqs[m].at[idxs[m]], rows[m], sem) for m in range(4)
    ]
    for c in copies:
      c.wait()

    dnums = lax.GatherDimensionNumbers(
        offset_dims=(), collapsed_slice_dims=(0,), start_index_map=(0,))

    def _lane_bcast(v, idx2):
      return lax.gather(v, idx2, dnums, slice_sizes=(1,),
                        mode=lax.GatherScatterMode.PROMISE_IN_BOUNDS)

    def p_body(p, c2):
      pidx = jnp.full((_L, 1), p, jnp.int32)
      for m in range(4):
        cm = _CHANS[m]
        w0 = _lane_bcast(wvecs[m][0], pidx)
        w1 = _lane_bcast(wvecs[m][1], pidx)
        w2 = _lane_bcast(wvecs[m][2], pidx)
        w3 = _lane_bcast(wvecs[m][3], pidx)
        for j in range(cm // _L):
          out_buf[p, pl.ds(_OFFS[m] + j * _L, _L)] = (
              w0 * rows[m][p, pl.ds(0 * cm + j * _L, _L)]
              + w1 * rows[m][p, pl.ds(1 * cm + j * _L, _L)]
              + w2 * rows[m][p, pl.ds(2 * cm + j * _L, _L)]
              + w3 * rows[m][p, pl.ds(3 * cm + j * _L, _L)])
      return c2

    lax.fori_loop(0, _L, p_body, 0)
    pltpu.sync_copy(out_buf, out_hbm.at[pl.ds(gb, _L), :])
    return carry

  lax.fori_loop(0, n_blk, block_body, 0)


@functools.partial(jax.jit, static_argnums=(7,))
def _sc_call(tq0, tq1, tq2, tq3, xs, ys, zs, n):
  nblk = n // _L
  mesh = plsc.VectorSubcoreMesh(core_axis_name="c", subcore_axis_name="s")
  scratch = [
      pltpu.VMEM((_L,), jnp.float32),
      pltpu.VMEM((_L,), jnp.float32),
      pltpu.VMEM((_L,), jnp.float32),
      pltpu.VMEM((_L,), jnp.int32),
      pltpu.VMEM((_L,), jnp.int32),
      pltpu.VMEM((_L,), jnp.int32),
      pltpu.VMEM((_L,), jnp.int32),
      pltpu.VMEM((_L, 4 * _CHANS[0]), jnp.float32),
      pltpu.VMEM((_L, 4 * _CHANS[1]), jnp.float32),
      pltpu.VMEM((_L, 4 * _CHANS[2]), jnp.float32),
      pltpu.VMEM((_L, 4 * _CHANS[3]), jnp.float32),
      pltpu.VMEM((_L, _FEAT_D), jnp.float32),
      pltpu.SemaphoreType.DMA,
  ]
  kfn = pl.kernel(
      functools.partial(_sc_body, nblk),
      out_type=jax.ShapeDtypeStruct((n, _FEAT_D), jnp.float32),
      mesh=mesh,
      scratch_types=scratch,
      compiler_params=pltpu.CompilerParams(use_tc_tiling_on_sc=False),
  )
  return kfn(tq0, tq1, tq2, tq3, xs, ys, zs)


def kernel(feat0, feat1, feat2, feat3, points):
  pts = points[0]
  n = pts.shape[0]
  tqs = []
  for feat, s in zip((feat0, feat1, feat2, feat3), _SIZES):
    fm = jnp.transpose(feat[0], (1, 2, 0))
    fp = jnp.pad(fm, ((0, 1), (0, 1), (0, 0)))
    quad = jnp.concatenate(
        [fp[:s, :s], fp[:s, 1:], fp[1:, :s], fp[1:, 1:]], axis=2)
    tqs.append(quad.reshape(s * s, 4 * fm.shape[2]))
  npad = -n % _L
  ppts = jnp.pad(pts, ((0, npad), (0, 0)), constant_values=1.0) if npad else pts
  xs, ys, zs = ppts[:, 0], ppts[:, 1], ppts[:, 2]
  feats = _sc_call(tqs[0], tqs[1], tqs[2], tqs[3], xs, ys, zs, n + npad)
  if npad:
    feats = feats[:n]
  return jnp.concatenate([pts, feats], axis=1)[None]


# speedup vs baseline: 3.5204x; 1.0326x over previous
import functools

import jax
import jax.numpy as jnp
from jax import lax
from jax.experimental import pallas as pl
from jax.experimental.pallas import tpu as pltpu
from jax.experimental.pallas import tpu_sc as plsc

_NC = 2
_NS = 16
_NW = _NC * _NS
_L = 16

_SIZES = (56, 28, 14, 7)
_CHANS = (64, 128, 256, 512)
_FEAT_D = 960
_OFFS = (0, 64, 192, 448)
_SLAB_BLK = 64
_SLAB_PTS = _SLAB_BLK * _L


def _sc_body(nblk_total, tq0, tq1, tq2, tq3, xs_hbm, ys_hbm, zs_hbm, out_hbm,
             xs_sl, ys_sl, zs_sl,
             ixa0, ixa1, ixa2, ixa3, ixb0, ixb1, ixb2, ixb3,
             ra0, ra1, ra2, ra3, rb0, rb1, rb2, rb3,
             oba, obb, gsa, gsb, osa, osb):
  tqs = (tq0, tq1, tq2, tq3)
  ixs = ((ixa0, ixa1, ixa2, ixa3), (ixb0, ixb1, ixb2, ixb3))
  rows = ((ra0, ra1, ra2, ra3), (rb0, rb1, rb2, rb3))
  obs = (oba, obb)
  gsems = (gsa, gsb)
  osems = (osa, osb)

  wid = lax.axis_index("s") * _NC + lax.axis_index("c")
  npair_total = nblk_total // 2
  q, r = npair_total // _NW, npair_total % _NW
  base_blk = 2 * (wid * q + jnp.minimum(wid, r))
  n_pair = q + jnp.where(wid < r, 1, 0)
  n_blk = 2 * n_pair

  lanes = lax.iota(jnp.int32, _L)
  zero = jnp.zeros((_L,), jnp.float32)
  himask = jnp.full((_L,), -65536, jnp.int32)
  shift16 = jnp.full((_L,), 16, jnp.int32)

  dnums = lax.GatherDimensionNumbers(
      offset_dims=(), collapsed_slice_dims=(0,), start_index_map=(0,))

  def _lane_bcast(v, idx2):
    return lax.gather(v, idx2, dnums, slice_sizes=(1,),
                      mode=lax.GatherScatterMode.PROMISE_IN_BOUNDS)

  def load_slab(l):
    start = (base_blk + l) * _L
    pltpu.sync_copy(xs_hbm.at[pl.ds(start, _SLAB_PTS)], xs_sl)
    pltpu.sync_copy(ys_hbm.at[pl.ds(start, _SLAB_PTS)], ys_sl)
    pltpu.sync_copy(zs_hbm.at[pl.ds(start, _SLAB_PTS)], zs_sl)

  def compute_iw(l, ix):
    o = (l % _SLAB_BLK) * _L
    xv = xs_sl[pl.ds(o, _L)]
    yv = ys_sl[pl.ds(o, _L)]
    zv = zs_sl[pl.ds(o, _L)]
    h = 248.0 * (yv / zv) + 111.5
    w = 248.0 * (xv / (-zv)) + 111.5
    h = jnp.clip(h, 0.0, 223.0)
    w = jnp.clip(w, 0.0, 223.0)
    wvecs = []
    for m in range(4):
      s = _SIZES[m]
      x = h * (s / 224.0)
      y = w * (s / 224.0)
      x1i = x.astype(jnp.int32)
      x1f = x1i.astype(jnp.float32)
      x2f = jnp.minimum(jnp.where(x > x1f, x1f + 1.0, x1f), float(s - 1))
      y1i = y.astype(jnp.int32)
      y1f = y1i.astype(jnp.float32)
      y2f = jnp.minimum(jnp.where(y > y1f, y1f + 1.0, y1f), float(s - 1))
      dx0 = x2f - x
      dx1 = x - x1f
      dy0 = y2f - y
      dy1 = y - y1f
      w11 = dx0 * dy0
      w12 = dx0 * dy1
      w21 = dx1 * dy0
      w22 = dx1 * dy1
      ysame = y2f == y1f
      xsame = x2f == x1f
      a0 = w11 + jnp.where(ysame, w12, zero)
      a1 = jnp.where(ysame, zero, w12)
      a2 = w21 + jnp.where(ysame, w22, zero)
      a3 = jnp.where(ysame, zero, w22)
      wvecs.extend((
          a0 + jnp.where(xsame, a2, zero),
          a1 + jnp.where(xsame, a3, zero),
          jnp.where(xsame, zero, a2),
          jnp.where(xsame, zero, a3),
      ))
      ix[m][...] = x1i * s + y1i
    return tuple(wvecs)

  def fire_gathers(ix, rw, sem):
    for m in range(4):
      pltpu.async_copy(tqs[m].at[ix[m]], rw[m], sem)

  def drain_gathers(ix, rw, sem):
    for m in range(4):
      pltpu.make_async_copy(tqs[m].at[ix[m]], rw[m], sem).wait()

  def combine(wv, rw, ob):
    def p_body(p, c2):
      pidx = jnp.full((_L, 1), p, jnp.int32)
      for m in range(4):
        cm = _CHANS[m]
        hw = cm // 2
        w0 = _lane_bcast(wv[4 * m + 0], pidx)
        w1 = _lane_bcast(wv[4 * m + 1], pidx)
        w2 = _lane_bcast(wv[4 * m + 2], pidx)
        w3 = _lane_bcast(wv[4 * m + 3], pidx)
        for g in range(cm // 32):
          acc_lo = zero
          acc_hi = zero
          for c, wc in ((0, w0), (1, w1), (2, w2), (3, w3)):
            qw = rw[m][p, pl.ds(c * hw + g * _L, _L)]
            lo = lax.bitcast_convert_type(lax.shift_left(qw, shift16),
                                          jnp.float32)
            hi = lax.bitcast_convert_type(lax.bitwise_and(qw, himask),
                                          jnp.float32)
            acc_lo = acc_lo + wc * lo
            acc_hi = acc_hi + wc * hi
          ob[p, pl.ds(_OFFS[m] + g * 32, _L)] = acc_lo
          ob[p, pl.ds(_OFFS[m] + g * 32 + _L, _L)] = acc_hi
      return c2

    lax.fori_loop(0, _L, p_body, 0)

  def out_dst(l):
    return out_hbm.at[pl.ds((base_blk + l) * _L, _L), :]

  load_slab(0)
  wv0 = compute_iw(0, ixs[0])
  fire_gathers(ixs[0], rows[0], gsems[0])

  def half(i, wv, cur, nxt):
    has_next = i + 1 < n_blk

    @pl.when(jnp.logical_and(has_next, (i + 1) % _SLAB_BLK == 0))
    def _():
      load_slab(i + 1)

    wv_next = compute_iw(i + 1, ixs[nxt])

    @pl.when(has_next)
    def _():
      fire_gathers(ixs[nxt], rows[nxt], gsems[nxt])

    drain_gathers(ixs[cur], rows[cur], gsems[cur])

    @pl.when(i >= 2)
    def _():
      pltpu.make_async_copy(obs[cur], out_dst(i), osems[cur]).wait()

    combine(wv, rows[cur], obs[cur])
    pltpu.async_copy(obs[cur], out_dst(i), osems[cur])
    return wv_next

  def body(i2, wv):
    wv1 = half(2 * i2, wv, 0, 1)
    return half(2 * i2 + 1, wv1, 1, 0)

  lax.fori_loop(0, n_pair, body, wv0)

  pltpu.make_async_copy(obs[0], out_dst(n_blk - 1), osems[0]).wait()
  pltpu.make_async_copy(obs[1], out_dst(n_blk - 2), osems[1]).wait()


@functools.partial(jax.jit, static_argnums=(7,))
def _sc_call(tq0, tq1, tq2, tq3, xs, ys, zs, n):
  nblk = n // _L
  mesh = plsc.VectorSubcoreMesh(core_axis_name="c", subcore_axis_name="s")
  scratch = (
      [pltpu.VMEM((_SLAB_PTS,), jnp.float32)] * 3
      + [pltpu.VMEM((_L,), jnp.int32)] * 8
      + [pltpu.VMEM((_L, 2 * c), jnp.int32) for c in _CHANS] * 2
      + [pltpu.VMEM((_L, _FEAT_D), jnp.float32)] * 2
      + [pltpu.SemaphoreType.DMA] * 4
  )
  kfn = pl.kernel(
      functools.partial(_sc_body, nblk),
      out_type=jax.ShapeDtypeStruct((n, _FEAT_D), jnp.float32),
      mesh=mesh,
      scratch_types=scratch,
      compiler_params=pltpu.CompilerParams(use_tc_tiling_on_sc=False),
  )
  return kfn(tq0, tq1, tq2, tq3, xs, ys, zs)


def kernel(feat0, feat1, feat2, feat3, points):
  pts = points[0]
  n = pts.shape[0]
  tqs = []
  for feat, s, c in zip((feat0, feat1, feat2, feat3), _SIZES, _CHANS):
    fm = jnp.transpose(feat[0], (1, 2, 0))
    fp = jnp.pad(fm, ((0, 1), (0, 1), (0, 0)))
    quad = jnp.concatenate(
        [fp[:s, :s], fp[:s, 1:], fp[1:, :s], fp[1:, 1:]], axis=2)
    quad = quad.reshape(s * s, 4 * c)
    qi = quad.reshape(s * s, 4, c // 32, 2, _L).transpose(0, 1, 2, 4, 3)
    qb = qi.astype(jnp.bfloat16).reshape(s * s, 2 * c, 2)
    tqs.append(lax.bitcast_convert_type(qb, jnp.int32))
  npad = -n % (2 * _L)
  ppts = jnp.pad(pts, ((0, npad), (0, 0)), constant_values=1.0) if npad else pts
  xs = jnp.pad(ppts[:, 0], (0, _SLAB_PTS * 4))
  ys = jnp.pad(ppts[:, 1], (0, _SLAB_PTS * 4))
  zs = jnp.pad(ppts[:, 2], (0, _SLAB_PTS * 4))
  feats = _sc_call(tqs[0], tqs[1], tqs[2], tqs[3], xs, ys, zs, n + npad)
  if npad:
    feats = feats[:n]
  return jnp.concatenate([pts, feats], axis=1)[None]


# speedup vs baseline: 4.4303x; 1.2585x over previous
import functools

import jax
import jax.numpy as jnp
from jax import lax
from jax.experimental import pallas as pl
from jax.experimental.pallas import tpu as pltpu
from jax.experimental.pallas import tpu_sc as plsc

_NC = 2
_NS = 16
_NW = _NC * _NS
_L = 16

_SIZES = (56, 28, 14, 7)
_CHANS = (64, 128, 256, 512)
_FEAT_D = 960
_OFFS = (0, 64, 192, 448)
_SLAB_BLK = 64
_SLAB_PTS = _SLAB_BLK * _L


def _sc_body(nblk_total, tq0, tq1, tq2, tq3, xs_hbm, ys_hbm, zs_hbm, out_hbm,
             xs_sl, ys_sl, zs_sl,
             ixa0, ixa1, ixa2, ixa3, ixb0, ixb1, ixb2, ixb3,
             ra0, ra1, ra2, ra3, rb0, rb1, rb2, rb3,
             oba, obb, zb, gsa, gsb, osa, osb):
  tqs = (tq0, tq1, tq2, tq3)
  ixs = ((ixa0, ixa1, ixa2, ixa3), (ixb0, ixb1, ixb2, ixb3))
  rows = ((ra0, ra1, ra2, ra3), (rb0, rb1, rb2, rb3))
  obs = (oba, obb)
  gsems = (gsa, gsb)
  osems = (osa, osb)

  wid = lax.axis_index("s") * _NC + lax.axis_index("c")
  npair_total = nblk_total // 2
  q, r = npair_total // _NW, npair_total % _NW
  base_blk = 2 * (wid * q + jnp.minimum(wid, r))
  n_pair = q + jnp.where(wid < r, 1, 0)
  n_blk = 2 * n_pair

  lanes = lax.iota(jnp.int32, _L)
  zero = jnp.zeros((_L,), jnp.float32)
  himask = jnp.full((_L,), -65536, jnp.int32)
  shift16 = jnp.full((_L,), 16, jnp.int32)

  dnums = lax.GatherDimensionNumbers(
      offset_dims=(), collapsed_slice_dims=(0,), start_index_map=(0,))

  def _lane_bcast(v, idx2):
    return lax.gather(v, idx2, dnums, slice_sizes=(1,),
                      mode=lax.GatherScatterMode.PROMISE_IN_BOUNDS)

  def load_slab(l):
    start = (base_blk + l) * _L
    pltpu.sync_copy(xs_hbm.at[pl.ds(start, _SLAB_PTS)], xs_sl)
    pltpu.sync_copy(ys_hbm.at[pl.ds(start, _SLAB_PTS)], ys_sl)
    pltpu.sync_copy(zs_hbm.at[pl.ds(start, _SLAB_PTS)], zs_sl)

  def compute_iw(l, ix):
    o = (l % _SLAB_BLK) * _L
    xv = xs_sl[pl.ds(o, _L)]
    yv = ys_sl[pl.ds(o, _L)]
    zv = zs_sl[pl.ds(o, _L)]
    h = 248.0 * (yv / zv) + 111.5
    w = 248.0 * (xv / (-zv)) + 111.5
    h = jnp.clip(h, 0.0, 223.0)
    w = jnp.clip(w, 0.0, 223.0)
    wvecs = []
    alive = []
    for m in range(4):
      s = _SIZES[m]
      x = h * (s / 224.0)
      y = w * (s / 224.0)
      x1i = x.astype(jnp.int32)
      x1f = x1i.astype(jnp.float32)
      x2f = jnp.minimum(jnp.where(x > x1f, x1f + 1.0, x1f), float(s - 1))
      y1i = y.astype(jnp.int32)
      y1f = y1i.astype(jnp.float32)
      y2f = jnp.minimum(jnp.where(y > y1f, y1f + 1.0, y1f), float(s - 1))
      dx0 = x2f - x
      dx1 = x - x1f
      dy0 = y2f - y
      dy1 = y - y1f
      w11 = dx0 * dy0
      w12 = dx0 * dy1
      w21 = dx1 * dy0
      w22 = dx1 * dy1
      ysame = y2f == y1f
      xsame = x2f == x1f
      a0 = w11 + jnp.where(ysame, w12, zero)
      a1 = jnp.where(ysame, zero, w12)
      a2 = w21 + jnp.where(ysame, w22, zero)
      a3 = jnp.where(ysame, zero, w22)
      b0 = a0 + jnp.where(xsame, a2, zero)
      b1 = a1 + jnp.where(xsame, a3, zero)
      b2 = jnp.where(xsame, zero, a2)
      b3 = jnp.where(xsame, zero, a3)
      wvecs.extend((b0, b1, b2, b3))
      t = jnp.abs(b0) + jnp.abs(b1) + jnp.abs(b2) + jnp.abs(b3)
      for k in (8, 4, 2, 1):
        t = t + _lane_bcast(t, ((lanes + k) & 15)[:, None])
      alive.append(lax.reshape(lax.slice(t, (0,), (1,)), ()) > 0.0)
      ix[m][...] = x1i * s + y1i
    return tuple(wvecs), tuple(alive)

  def fire_gathers(ix, rw, sem, alive, gate):
    for m in range(4):
      @pl.when(jnp.logical_and(gate, alive[m]))
      def _(m=m):
        pltpu.async_copy(tqs[m].at[ix[m]], rw[m], sem)

  def drain_gathers(ix, rw, sem, alive):
    for m in range(4):
      @pl.when(alive[m])
      def _(m=m):
        pltpu.make_async_copy(tqs[m].at[ix[m]], rw[m], sem).wait()

  def combine(wv, alive, rw, ob):
    for m in range(4):
      cm = _CHANS[m]
      hw = cm // 2

      @pl.when(alive[m])
      def _(m=m, cm=cm, hw=hw):
        def p_body(p, c2):
          pidx = jnp.full((_L, 1), p, jnp.int32)
          w0 = _lane_bcast(wv[4 * m + 0], pidx)
          w1 = _lane_bcast(wv[4 * m + 1], pidx)
          w2 = _lane_bcast(wv[4 * m + 2], pidx)
          w3 = _lane_bcast(wv[4 * m + 3], pidx)
          for g in range(cm // 32):
            acc_lo = zero
            acc_hi = zero
            for c, wc in ((0, w0), (1, w1), (2, w2), (3, w3)):
              qw = rw[m][p, pl.ds(c * hw + g * _L, _L)]
              lo = lax.bitcast_convert_type(lax.shift_left(qw, shift16),
                                            jnp.float32)
              hi = lax.bitcast_convert_type(lax.bitwise_and(qw, himask),
                                            jnp.float32)
              acc_lo = acc_lo + wc * lo
              acc_hi = acc_hi + wc * hi
            ob[p, pl.ds(_OFFS[m] + g * 32, _L)] = acc_lo
            ob[p, pl.ds(_OFFS[m] + g * 32 + _L, _L)] = acc_hi
          return c2

        lax.fori_loop(0, _L, p_body, 0, unroll=2)

  def write_out(l, alive, ob, sem):
    gb = (base_blk + l) * _L
    for m in range(4):
      cm = _CHANS[m]
      off = _OFFS[m]
      dst = out_hbm.at[pl.ds(gb, _L), pl.ds(off, cm)]

      @pl.when(alive[m])
      def _(dst=dst, off=off, cm=cm):
        pltpu.async_copy(ob.at[:, pl.ds(off, cm)], dst, sem)

      @pl.when(jnp.logical_not(alive[m]))
      def _(dst=dst, cm=cm):
        pltpu.async_copy(zb.at[:, pl.ds(0, cm)], dst, sem)

  def drain_out(l, ob, sem):
    gb = (base_blk + l) * _L
    for m in range(4):
      cm = _CHANS[m]
      off = _OFFS[m]
      pltpu.make_async_copy(
          ob.at[:, pl.ds(off, cm)],
          out_hbm.at[pl.ds(gb, _L), pl.ds(off, cm)], sem).wait()

  def zfill(p, c2):
    for g in range(512 // _L):
      zb[p, pl.ds(g * _L, _L)] = zero
    return c2

  lax.fori_loop(0, _L, zfill, 0)
  load_slab(0)
  wv0, al0 = compute_iw(0, ixs[0])
  fire_gathers(ixs[0], rows[0], gsems[0], al0, True)

  def half(i, wv, alive, cur, nxt):
    has_next = i + 1 < n_blk

    @pl.when(jnp.logical_and(has_next, (i + 1) % _SLAB_BLK == 0))
    def _():
      load_slab(i + 1)

    wv_next, al_next = compute_iw(i + 1, ixs[nxt])
    fire_gathers(ixs[nxt], rows[nxt], gsems[nxt], al_next, has_next)
    drain_gathers(ixs[cur], rows[cur], gsems[cur], alive)

    @pl.when(i >= 2)
    def _():
      drain_out(i, obs[cur], osems[cur])

    combine(wv, alive, rows[cur], obs[cur])
    write_out(i, alive, obs[cur], osems[cur])
    return wv_next, al_next

  def body(i2, carry):
    wv, alive = carry
    wv1, al1 = half(2 * i2, wv, alive, 0, 1)
    return half(2 * i2 + 1, wv1, al1, 1, 0)

  lax.fori_loop(0, n_pair, body, (wv0, al0))

  drain_out(n_blk - 1, obs[0], osems[0])
  drain_out(n_blk - 2, obs[1], osems[1])


@functools.partial(jax.jit, static_argnums=(7,))
def _sc_call(tq0, tq1, tq2, tq3, xs, ys, zs, n):
  nblk = n // _L
  mesh = plsc.VectorSubcoreMesh(core_axis_name="c", subcore_axis_name="s")
  scratch = (
      [pltpu.VMEM((_SLAB_PTS,), jnp.float32)] * 3
      + [pltpu.VMEM((_L,), jnp.int32)] * 8
      + [pltpu.VMEM((_L, 2 * c), jnp.int32) for c in _CHANS] * 2
      + [pltpu.VMEM((_L, _FEAT_D), jnp.float32)] * 2
      + [pltpu.VMEM((_L, 512), jnp.float32)]
      + [pltpu.SemaphoreType.DMA] * 4
  )
  kfn = pl.kernel(
      functools.partial(_sc_body, nblk),
      out_type=jax.ShapeDtypeStruct((n, _FEAT_D), jnp.float32),
      mesh=mesh,
      scratch_types=scratch,
      compiler_params=pltpu.CompilerParams(use_tc_tiling_on_sc=False),
  )
  return kfn(tq0, tq1, tq2, tq3, xs, ys, zs)


def kernel(feat0, feat1, feat2, feat3, points):
  pts = points[0]
  n = pts.shape[0]
  tqs = []
  for feat, s, c in zip((feat0, feat1, feat2, feat3), _SIZES, _CHANS):
    fm = jnp.transpose(feat[0], (1, 2, 0))
    fp = jnp.pad(fm, ((0, 1), (0, 1), (0, 0)))
    quad = jnp.concatenate(
        [fp[:s, :s], fp[:s, 1:], fp[1:, :s], fp[1:, 1:]], axis=2)
    quad = quad.reshape(s * s, 4 * c)
    qi = quad.reshape(s * s, 4, c // 32, 2, _L).transpose(0, 1, 2, 4, 3)
    qb = qi.astype(jnp.bfloat16).reshape(s * s, 2 * c, 2)
    tqs.append(lax.bitcast_convert_type(qb, jnp.int32))
  npad = -n % (2 * _L)
  ppts = jnp.pad(pts, ((0, npad), (0, 0)), constant_values=1.0) if npad else pts
  xs = jnp.pad(ppts[:, 0], (0, _SLAB_PTS * 4))
  ys = jnp.pad(ppts[:, 1], (0, _SLAB_PTS * 4))
  zs = jnp.pad(ppts[:, 2], (0, _SLAB_PTS * 4))
  feats = _sc_call(tqs[0], tqs[1], tqs[2], tqs[3], xs, ys, zs, n + npad)
  if npad:
    feats = feats[:n]
  out = (jnp.pad(feats, ((0, 0), (3, 0)))
         + jnp.pad(pts, ((0, 0), (0, _FEAT_D))))
  return out[None]


# speedup vs baseline: 4.4348x; 1.0010x over previous
import functools

import jax
import jax.numpy as jnp
from jax import lax
from jax.experimental import pallas as pl
from jax.experimental.pallas import tpu as pltpu
from jax.experimental.pallas import tpu_sc as plsc

_NC = 2
_NS = 16
_NW = _NC * _NS
_L = 16

_SIZES = (56, 28, 14, 7)
_CHANS = (64, 128, 256, 512)
_FEAT_D = 960
_OFFS = (0, 64, 192, 448)
_SLAB_BLK = 64
_SLAB_PTS = _SLAB_BLK * _L


def _sc_body(nblk_total, tq0, tq1, tq2, tq3, xs_hbm, ys_hbm, zs_hbm, out_hbm,
             xs_sl, ys_sl, zs_sl,
             ixa0, ixa1, ixa2, ixa3, ixb0, ixb1, ixb2, ixb3,
             ra0, ra1, ra2, ra3, rb0, rb1, rb2, rb3,
             oba, obb, zb, gsa, gsb, osa, osb):
  tqs = (tq0, tq1, tq2, tq3)
  ixs = ((ixa0, ixa1, ixa2, ixa3), (ixb0, ixb1, ixb2, ixb3))
  rows = ((ra0, ra1, ra2, ra3), (rb0, rb1, rb2, rb3))
  obs = (oba, obb)
  gsems = (gsa, gsb)
  osems = (osa, osb)

  wid = lax.axis_index("s") * _NC + lax.axis_index("c")
  npair_total = nblk_total // 2
  q, r = npair_total // _NW, npair_total % _NW
  base_blk = 2 * (wid * q + jnp.minimum(wid, r))
  n_pair = q + jnp.where(wid < r, 1, 0)
  n_blk = 2 * n_pair

  lanes = lax.iota(jnp.int32, _L)
  zero = jnp.zeros((_L,), jnp.float32)
  himask = jnp.full((_L,), -65536, jnp.int32)
  shift16 = jnp.full((_L,), 16, jnp.int32)

  dnums = lax.GatherDimensionNumbers(
      offset_dims=(), collapsed_slice_dims=(0,), start_index_map=(0,))

  def _lane_bcast(v, idx2):
    return lax.gather(v, idx2, dnums, slice_sizes=(1,),
                      mode=lax.GatherScatterMode.PROMISE_IN_BOUNDS)

  def load_slab(l):
    start = (base_blk + l) * _L
    pltpu.sync_copy(xs_hbm.at[pl.ds(start, _SLAB_PTS)], xs_sl)
    pltpu.sync_copy(ys_hbm.at[pl.ds(start, _SLAB_PTS)], ys_sl)
    pltpu.sync_copy(zs_hbm.at[pl.ds(start, _SLAB_PTS)], zs_sl)

  def compute_iw(l, ix):
    o = (l % _SLAB_BLK) * _L
    xv = xs_sl[pl.ds(o, _L)]
    yv = ys_sl[pl.ds(o, _L)]
    zv = zs_sl[pl.ds(o, _L)]
    h = 248.0 * (yv / zv) + 111.5
    w = 248.0 * (xv / (-zv)) + 111.5
    h = jnp.clip(h, 0.0, 223.0)
    w = jnp.clip(w, 0.0, 223.0)
    wvecs = []
    alive = []
    for m in range(4):
      s = _SIZES[m]
      x = h * (s / 224.0)
      y = w * (s / 224.0)
      x1i = x.astype(jnp.int32)
      x1f = x1i.astype(jnp.float32)
      x2f = jnp.minimum(jnp.where(x > x1f, x1f + 1.0, x1f), float(s - 1))
      y1i = y.astype(jnp.int32)
      y1f = y1i.astype(jnp.float32)
      y2f = jnp.minimum(jnp.where(y > y1f, y1f + 1.0, y1f), float(s - 1))
      dx0 = x2f - x
      dx1 = x - x1f
      dy0 = y2f - y
      dy1 = y - y1f
      w11 = dx0 * dy0
      w12 = dx0 * dy1
      w21 = dx1 * dy0
      w22 = dx1 * dy1
      ysame = y2f == y1f
      xsame = x2f == x1f
      a0 = w11 + jnp.where(ysame, w12, zero)
      a1 = jnp.where(ysame, zero, w12)
      a2 = w21 + jnp.where(ysame, w22, zero)
      a3 = jnp.where(ysame, zero, w22)
      b0 = a0 + jnp.where(xsame, a2, zero)
      b1 = a1 + jnp.where(xsame, a3, zero)
      b2 = jnp.where(xsame, zero, a2)
      b3 = jnp.where(xsame, zero, a3)
      wvecs.extend((b0, b1, b2, b3))
      t = jnp.abs(b0) + jnp.abs(b1) + jnp.abs(b2) + jnp.abs(b3)
      for k in (8, 4, 2, 1):
        t = t + _lane_bcast(t, ((lanes + k) & 15)[:, None])
      alive.append(lax.reshape(lax.slice(t, (0,), (1,)), ()) > 0.0)
      ix[m][...] = x1i * s + y1i
    return tuple(wvecs), tuple(alive)

  def fire_gathers(ix, rw, sem, alive, gate):
    for m in range(4):
      @pl.when(jnp.logical_and(gate, alive[m]))
      def _(m=m):
        pltpu.async_copy(tqs[m].at[ix[m]], rw[m], sem)

  def drain_gathers(ix, rw, sem, alive):
    for m in range(4):
      @pl.when(alive[m])
      def _(m=m):
        pltpu.make_async_copy(tqs[m].at[ix[m]], rw[m], sem).wait()

  def combine(wv, alive, rw, ob):
    for m in range(4):
      cm = _CHANS[m]
      hw = cm // 2

      @pl.when(alive[m])
      def _(m=m, cm=cm, hw=hw):
        def p_body(p, c2):
          pidx = jnp.full((_L, 1), p, jnp.int32)
          w0 = _lane_bcast(wv[4 * m + 0], pidx)
          w1 = _lane_bcast(wv[4 * m + 1], pidx)
          w2 = _lane_bcast(wv[4 * m + 2], pidx)
          w3 = _lane_bcast(wv[4 * m + 3], pidx)
          for g in range(cm // 32):
            acc_lo = zero
            acc_hi = zero
            for c, wc in ((0, w0), (1, w1), (2, w2), (3, w3)):
              qw = rw[m][p, pl.ds(c * hw + g * _L, _L)]
              lo = lax.bitcast_convert_type(lax.shift_left(qw, shift16),
                                            jnp.float32)
              hi = lax.bitcast_convert_type(lax.bitwise_and(qw, himask),
                                            jnp.float32)
              acc_lo = acc_lo + wc * lo
              acc_hi = acc_hi + wc * hi
            ob[p, pl.ds(_OFFS[m] + g * 32, _L)] = acc_lo
            ob[p, pl.ds(_OFFS[m] + g * 32 + _L, _L)] = acc_hi
          return c2

        lax.fori_loop(0, _L, p_body, 0, unroll=2)

  def write_out(l, alive, ob, sem):
    gb = (base_blk + l) * _L
    for m in range(4):
      cm = _CHANS[m]
      off = _OFFS[m]
      dst = out_hbm.at[pl.ds(gb, _L), pl.ds(off, cm)]

      @pl.when(alive[m])
      def _(dst=dst, off=off, cm=cm):
        pltpu.async_copy(ob.at[:, pl.ds(off, cm)], dst, sem)

      @pl.when(jnp.logical_not(alive[m]))
      def _(dst=dst, cm=cm):
        pltpu.async_copy(zb.at[:, pl.ds(0, cm)], dst, sem)

  def drain_out(l, ob, sem):
    gb = (base_blk + l) * _L
    for m in range(4):
      cm = _CHANS[m]
      off = _OFFS[m]
      pltpu.make_async_copy(
          ob.at[:, pl.ds(off, cm)],
          out_hbm.at[pl.ds(gb, _L), pl.ds(off, cm)], sem).wait()

  def zfill(p, c2):
    for g in range(512 // _L):
      zb[p, pl.ds(g * _L, _L)] = zero
    return c2

  lax.fori_loop(0, _L, zfill, 0)
  load_slab(0)
  wv0, al0 = compute_iw(0, ixs[0])
  fire_gathers(ixs[0], rows[0], gsems[0], al0, True)

  def half(i, wv, alive, cur, nxt):
    has_next = i + 1 < n_blk

    @pl.when(jnp.logical_and(has_next, (i + 1) % _SLAB_BLK == 0))
    def _():
      load_slab(i + 1)

    wv_next, al_next = compute_iw(i + 1, ixs[nxt])
    fire_gathers(ixs[nxt], rows[nxt], gsems[nxt], al_next, has_next)
    drain_gathers(ixs[cur], rows[cur], gsems[cur], alive)

    @pl.when(i >= 2)
    def _():
      drain_out(i, obs[cur], osems[cur])

    combine(wv, alive, rows[cur], obs[cur])
    write_out(i, alive, obs[cur], osems[cur])
    return wv_next, al_next

  def body(i2, carry):
    wv, alive = carry
    wv1, al1 = half(2 * i2, wv, alive, 0, 1)
    return half(2 * i2 + 1, wv1, al1, 1, 0)

  lax.fori_loop(0, n_pair, body, (wv0, al0))

  drain_out(n_blk - 1, obs[0], osems[0])
  drain_out(n_blk - 2, obs[1], osems[1])


@functools.partial(jax.jit, static_argnums=(7,))
def _sc_call(tq0, tq1, tq2, tq3, xs, ys, zs, n):
  nblk = n // _L
  mesh = plsc.VectorSubcoreMesh(core_axis_name="c", subcore_axis_name="s")
  scratch = (
      [pltpu.VMEM((_SLAB_PTS,), jnp.float32)] * 3
      + [pltpu.VMEM((_L,), jnp.int32)] * 8
      + [pltpu.VMEM((_L, 2 * c), jnp.int32) for c in _CHANS] * 2
      + [pltpu.VMEM((_L, _FEAT_D), jnp.float32)] * 2
      + [pltpu.VMEM((_L, 512), jnp.float32)]
      + [pltpu.SemaphoreType.DMA] * 4
  )
  kfn = pl.kernel(
      functools.partial(_sc_body, nblk),
      out_type=jax.ShapeDtypeStruct((n, _FEAT_D), jnp.float32),
      mesh=mesh,
      scratch_types=scratch,
      compiler_params=pltpu.CompilerParams(use_tc_tiling_on_sc=False),
  )
  return kfn(tq0, tq1, tq2, tq3, xs, ys, zs)


def kernel(feat0, feat1, feat2, feat3, points):
  pts = points[0]
  n = pts.shape[0]
  tqs = []
  for feat, s, c in zip((feat0, feat1, feat2, feat3), _SIZES, _CHANS):
    fm = jnp.transpose(feat[0], (1, 2, 0))
    fp = jnp.pad(fm, ((0, 1), (0, 1), (0, 0)))
    quad = jnp.concatenate(
        [fp[:s, :s], fp[:s, 1:], fp[1:, :s], fp[1:, 1:]], axis=2)
    quad = quad.reshape(s * s, 4 * c)
    qi = quad.reshape(s * s, 4, c // 32, 2, _L).transpose(0, 1, 2, 4, 3)
    qb = qi.astype(jnp.bfloat16).reshape(s * s, 2 * c, 2)
    tqs.append(lax.bitcast_convert_type(qb, jnp.int32))
  npad = -n % (2 * _L)
  ppts = jnp.pad(pts, ((0, npad), (0, 0)), constant_values=1.0) if npad else pts
  xs = jnp.pad(ppts[:, 0], (0, _SLAB_PTS * 4))
  ys = jnp.pad(ppts[:, 1], (0, _SLAB_PTS * 4))
  zs = jnp.pad(ppts[:, 2], (0, _SLAB_PTS * 4))
  feats = _sc_call(tqs[0], tqs[1], tqs[2], tqs[3], xs, ys, zs, n + npad)
  out = _assemble(pts if npad == 0 else ppts, feats)
  if npad:
    out = out[:n]
  return out[None]


def _assemble_body(pts_ref, feats_ref, out_ref):
  out_ref[:, 0:3] = pts_ref[...]
  out_ref[:, 3:] = feats_ref[...]


def _assemble(pts, feats):
  n = pts.shape[0]
  blk = 2000
  while n % blk or blk % 8:
    blk //= 2
  grid = (n // blk,)
  return pl.pallas_call(
      _assemble_body,
      grid=grid,
      in_specs=[
          pl.BlockSpec((blk, 3), lambda i: (i, 0)),
          pl.BlockSpec((blk, _FEAT_D), lambda i: (i, 0)),
      ],
      out_specs=pl.BlockSpec((blk, 3 + _FEAT_D), lambda i: (i, 0)),
      out_shape=jax.ShapeDtypeStruct((n, 3 + _FEAT_D), jnp.float32),
  )(pts, feats)
